# baseline (speedup 1.0000x reference)
import functools

import jax
import jax.numpy as jnp
from jax import lax
from jax.experimental import pallas as pl
from jax.experimental.pallas import tpu as pltpu
from jax.experimental.pallas import tpu_sc as plsc

N_NODES = 10000
N_EDGES = 320000
D = 128
DH = D // 2

NC, NS = 2, 16
CH = 128
EPT = 20480
NCHUNK = EPT // CH
E_PAD = EPT * NS
R = 10240
RPT = R // NS


CH2 = 64
NCHUNK2 = 318
EPT2 = CH2 * NCHUNK2
E_PAD2 = EPT2 * NS


def _sc_body2(tab2_hbm, src_hbm, dst_hbm, z2,
              agg_hbm,
              tab_sh, acc_sh, src_v, dst_v, b0, b1,
              g0, g1, s0, s1):
    bufs = (b0, b1)
    gsem = (g0, g1)
    ssem = (s0, s1)

    cid = lax.axis_index("c")
    sid = lax.axis_index("s")
    row0 = sid * RPT

    pltpu.sync_copy(z2, bufs[0])
    for k in range(RPT // CH2):
        pltpu.sync_copy(bufs[0], acc_sh.at[pl.ds(row0 + k * CH2, CH2)])
    for k in range(RPT // CH2):
        r = row0 + k * CH2
        pltpu.sync_copy(tab2_hbm.at[cid, pl.ds(r, CH2)], bufs[1])
        pltpu.sync_copy(bufs[1], tab_sh.at[pl.ds(r, CH2)])

    pltpu.sync_copy(src_hbm.at[sid], src_v)
    pltpu.sync_copy(dst_hbm.at[sid], dst_v)

    plsc.subcore_barrier()

    for b in range(2):
        pltpu.async_copy(tab_sh.at[src_v.at[b]], bufs[b], gsem[b])

    NR = NCHUNK2 // 2

    def round_body(g, carry):
        for b in range(2):
            j = 2 * g + b
            pltpu.make_async_copy(
                tab_sh.at[src_v.at[j]], bufs[b], gsem[b]).wait()
            pltpu.async_copy(bufs[b], acc_sh.at[dst_v.at[j]], ssem[b],
                             add=True)
        for b in range(2):
            j = 2 * g + b
            pltpu.make_async_copy(
                bufs[b], acc_sh.at[dst_v.at[j]], ssem[b]).wait()
            pltpu.async_copy(tab_sh.at[src_v.at[j + 2]], bufs[b], gsem[b])
        return carry

    lax.fori_loop(0, NR - 1, round_body, 0)
    for b in range(2):
        j = (NR - 1) * 2 + b
        pltpu.make_async_copy(
            tab_sh.at[src_v.at[j]], bufs[b], gsem[b]).wait()
        pltpu.async_copy(bufs[b], acc_sh.at[dst_v.at[j]], ssem[b], add=True)
    for b in range(2):
        j = (NR - 1) * 2 + b
        pltpu.make_async_copy(
            bufs[b], acc_sh.at[dst_v.at[j]], ssem[b]).wait()

    plsc.subcore_barrier()

    nwb = RPT // CH2
    for k in range(nwb):
        b = k % 2
        r = row0 + k * CH2
        if k >= 2:
            rp = row0 + (k - 2) * CH2
            pltpu.make_async_copy(
                bufs[b], agg_hbm.at[pl.ds(rp, CH2), cid], gsem[b]).wait()
        pltpu.sync_copy(acc_sh.at[pl.ds(r, CH2)], bufs[b])
        pltpu.async_copy(bufs[b], agg_hbm.at[pl.ds(r, CH2), cid], gsem[b])
    for k in range(nwb - 2, nwb):
        b = k % 2
        r = row0 + k * CH2
        pltpu.make_async_copy(
            bufs[b], agg_hbm.at[pl.ds(r, CH2), cid], gsem[b]).wait()


def _make_sc_pass2():
    mesh = plsc.VectorSubcoreMesh(core_axis_name="c", subcore_axis_name="s")
    return pl.kernel(
        _sc_body2,
        out_type=[jax.ShapeDtypeStruct((R, NC, DH), jnp.float32)],
        mesh=mesh,
        scratch_types=[
            pltpu.VMEM_SHARED((R, DH), jnp.float32),
            pltpu.VMEM_SHARED((R, DH), jnp.float32),
            pltpu.VMEM((NCHUNK2, CH2), jnp.int32),
            pltpu.VMEM((NCHUNK2, CH2), jnp.int32),
            pltpu.VMEM((CH2, DH), jnp.float32),
            pltpu.VMEM((CH2, DH), jnp.float32),
            pltpu.SemaphoreType.DMA, pltpu.SemaphoreType.DMA,
            pltpu.SemaphoreType.DMA, pltpu.SemaphoreType.DMA,
        ],
        compiler_params=pltpu.CompilerParams(use_tc_tiling_on_sc=False),
    )


def _sc_cnt_body(dst_hbm, z16, o16,
                 cnt_hbm,
                 cnt_sh, dst_v, onesb, cbuf, csem):
    cid = lax.axis_index("c")
    sid = lax.axis_index("s")
    row0 = sid * RPT

    pltpu.sync_copy(z16, cbuf)
    for k in range(RPT // 32):
        pltpu.sync_copy(cbuf, cnt_sh.at[pl.ds(row0 + k * 32, 32)])
    pltpu.sync_copy(o16, onesb)
    pltpu.sync_copy(dst_hbm.at[sid], dst_v)

    plsc.subcore_barrier()

    half = NCHUNK2 // 2
    base = cid * half

    def body(i, carry):
        pltpu.async_copy(onesb, cnt_sh.at[dst_v.at[base + i]], csem,
                         add=True)
        return carry

    lax.fori_loop(0, half, body, 0)

    def drain(i, carry):
        pltpu.make_async_copy(onesb, cnt_sh.at[dst_v.at[0]], csem).wait()
        return carry

    lax.fori_loop(0, half, drain, 0)

    plsc.subcore_barrier()

    for k in range(RPT // 32):
        r = row0 + k * 32
        pltpu.sync_copy(cnt_sh.at[pl.ds(r, 32)], cbuf)
        pltpu.sync_copy(cbuf, cnt_hbm.at[cid, pl.ds(r, 32)])


def _make_sc_cnt():
    mesh = plsc.VectorSubcoreMesh(core_axis_name="c", subcore_axis_name="s")
    return pl.kernel(
        _sc_cnt_body,
        out_type=[jax.ShapeDtypeStruct((NC, R, 16), jnp.float32)],
        mesh=mesh,
        scratch_types=[
            pltpu.VMEM_SHARED((R, 16), jnp.float32),
            pltpu.VMEM((NCHUNK2, CH2), jnp.int32),
            pltpu.VMEM((CH2, 16), jnp.float32),
            pltpu.VMEM((32, 16), jnp.float32),
            pltpu.SemaphoreType.DMA,
        ],
        compiler_params=pltpu.CompilerParams(use_tc_tiling_on_sc=False),
    )


def _tc_layer_body(agg_ref, cnt_ref, x_ref, wl_ref, wr_ref, b_ref, o_ref):
    c = cnt_ref[0, :, 0] + cnt_ref[1, :, 0]
    mean = agg_ref[...] / jnp.maximum(c, 1.0)[:, None]
    h = (jnp.dot(mean, wl_ref[...], preferred_element_type=jnp.float32)
         + jnp.dot(x_ref[...], wr_ref[...], preferred_element_type=jnp.float32)
         + b_ref[...])
    o_ref[...] = jnp.maximum(h, 0.0)


def _tc_layer(agg, cnt, x, wl, wr, b):
    blk = 640
    return pl.pallas_call(
        _tc_layer_body,
        grid=(R // blk,),
        in_specs=[
            pl.BlockSpec((blk, D), lambda i: (i, 0)),
            pl.BlockSpec((NC, blk, 16), lambda i: (0, i, 0)),
            pl.BlockSpec((blk, D), lambda i: (i, 0)),
            pl.BlockSpec((D, D), lambda i: (0, 0)),
            pl.BlockSpec((D, D), lambda i: (0, 0)),
            pl.BlockSpec((1, D), lambda i: (0, 0)),
        ],
        out_specs=pl.BlockSpec((blk, D), lambda i: (i, 0)),
        out_shape=jax.ShapeDtypeStruct((R, D), jnp.float32),
    )(agg, cnt, x, wl, wr, b)


def _tc_final_body(agg_ref, cnt_ref, h_ref, wl_ref, wr_ref, b_ref,
                   wf_ref, bf_ref, o_ref):
    c = cnt_ref[0, :, 0] + cnt_ref[1, :, 0]
    mean = agg_ref[...] / jnp.maximum(c, 1.0)[:, None]
    h2 = (jnp.dot(mean, wl_ref[...], preferred_element_type=jnp.float32)
          + jnp.dot(h_ref[...], wr_ref[...], preferred_element_type=jnp.float32)
          + b_ref[...])
    h2 = jnp.maximum(h2, 0.0)
    o_ref[...] = (jnp.dot(h2, wf_ref[...], preferred_element_type=jnp.float32)
                  + bf_ref[...])


def _tc_final(agg, cnt, h, wl, wr, b, wf_pad, bf_pad):
    blk = 640
    return pl.pallas_call(
        _tc_final_body,
        grid=(R // blk,),
        in_specs=[
            pl.BlockSpec((blk, D), lambda i: (i, 0)),
            pl.BlockSpec((NC, blk, 16), lambda i: (0, i, 0)),
            pl.BlockSpec((blk, D), lambda i: (i, 0)),
            pl.BlockSpec((D, D), lambda i: (0, 0)),
            pl.BlockSpec((D, D), lambda i: (0, 0)),
            pl.BlockSpec((1, D), lambda i: (0, 0)),
            pl.BlockSpec((D, D), lambda i: (0, 0)),
            pl.BlockSpec((1, D), lambda i: (0, 0)),
        ],
        out_specs=pl.BlockSpec((blk, D), lambda i: (i, 0)),
        out_shape=jax.ShapeDtypeStruct((R, D), jnp.float32),
    )(agg, cnt, h, wl, wr, b, wf_pad, bf_pad)


def kernel(x, edge_index, Wl1, Wr1, b1, Wl2, Wr2, b2, Wfc, bfc):
    src = edge_index[0].astype(jnp.int32)
    dst = edge_index[1].astype(jnp.int32)
    npad2 = E_PAD2 - N_EDGES
    pad_ar2 = jnp.arange(npad2, dtype=jnp.int32)
    srcp2 = jnp.concatenate(
        [src, pad_ar2 % N_NODES]).reshape(NS, NCHUNK2, CH2)
    dstp2 = jnp.concatenate(
        [dst, N_NODES + pad_ar2 % (R - N_NODES)]).reshape(NS, NCHUNK2, CH2)
    xp = jnp.zeros((R, D), jnp.float32).at[:N_NODES].set(x)
    x_split = xp.reshape(R, NC, DH).transpose(1, 0, 2)

    z2 = jnp.zeros((CH2, DH), jnp.float32)
    z16 = jnp.zeros((32, 16), jnp.float32)
    o16 = jnp.ones((CH2, 16), jnp.float32)

    sc_agg = _make_sc_pass2()
    sc_cnt = _make_sc_cnt()

    (cnt,) = sc_cnt(dstp2, z16, o16)
    (agg1,) = sc_agg(x_split, srcp2, dstp2, z2)
    h = _tc_layer(agg1.reshape(R, D), cnt, xp, Wl1, Wr1, b1.reshape(1, D))
    h_split = h.reshape(R, NC, DH).transpose(1, 0, 2)
    (agg2,) = sc_agg(h_split, srcp2, dstp2, z2)

    wf_pad = jnp.zeros((D, D), jnp.float32).at[:, :1].set(Wfc)
    bf_pad = jnp.broadcast_to(bfc.reshape(1, 1), (1, D))
    outp = _tc_final(agg2.reshape(R, D), cnt, h, Wl2, Wr2, b2.reshape(1, D),
                     wf_pad, bf_pad)
    return outp[:N_NODES, :1]


# speedup vs baseline: 1.1451x; 1.1451x over previous
import functools

import jax
import jax.numpy as jnp
from jax import lax
from jax.experimental import pallas as pl
from jax.experimental.pallas import tpu as pltpu
from jax.experimental.pallas import tpu_sc as plsc

N_NODES = 10000
N_EDGES = 320000
D = 128
DH = D // 2

NC, NS = 2, 16
CH = 128
EPT = 20480
NCHUNK = EPT // CH
E_PAD = EPT * NS
R = 10240
RPT = R // NS


NBUF = 4


def _sc_body(with_count, *refs):
    if with_count:
        (tab_hbm, src_hbm, dst_hbm, z64, z16, o16,
         agg_hbm, cnt_hbm,
         acc_sh, cnt_sh, src_v, dst_v, b0, b1, b2, b3, onesb, cbuf,
         g0, g1, g2, g3, s0, s1, s2, s3, csem) = refs
    else:
        (tab_hbm, src_hbm, dst_hbm, z64,
         agg_hbm,
         acc_sh, src_v, dst_v, b0, b1, b2, b3,
         g0, g1, g2, g3, s0, s1, s2, s3) = refs
    bufs = (b0, b1, b2, b3)
    gsem = (g0, g1, g2, g3)
    ssem = (s0, s1, s2, s3)

    cid = lax.axis_index("c")
    sid = lax.axis_index("s")
    row0 = sid * RPT

    pltpu.sync_copy(z64, bufs[0])
    for k in range(RPT // CH):
        pltpu.sync_copy(bufs[0], acc_sh.at[pl.ds(row0 + k * CH, CH)])
    if with_count:
        pltpu.sync_copy(z16, cbuf)
        for k in range(RPT // CH):
            pltpu.sync_copy(cbuf, cnt_sh.at[pl.ds(row0 + k * CH, CH)])
        pltpu.sync_copy(o16, onesb)

    pltpu.sync_copy(src_hbm.at[cid, sid], src_v)
    pltpu.sync_copy(dst_hbm.at[sid], dst_v)

    plsc.subcore_barrier()

    def count_scatter(j):
        if with_count:
            @pl.when((j < NCHUNK // 2) == (cid == 0))
            def _():
                pltpu.async_copy(onesb, cnt_sh.at[dst_v.at[j]], csem,
                                 add=True)

    for b in range(NBUF):
        pltpu.async_copy(tab_hbm.at[src_v.at[b]], bufs[b], gsem[b])

    NR = NCHUNK // NBUF

    def round_body(g, carry):
        for b in range(NBUF):
            j = g * NBUF + b
            pltpu.make_async_copy(
                tab_hbm.at[src_v.at[j]], bufs[b], gsem[b]).wait()
            pltpu.async_copy(bufs[b], acc_sh.at[dst_v.at[j]], ssem[b],
                             add=True)
            count_scatter(j)
        for b in range(NBUF):
            j = g * NBUF + b
            pltpu.make_async_copy(
                bufs[b], acc_sh.at[dst_v.at[j]], ssem[b]).wait()
            pltpu.async_copy(tab_hbm.at[src_v.at[j + NBUF]], bufs[b],
                             gsem[b])
        return carry

    lax.fori_loop(0, NR - 1, round_body, 0)
    for b in range(NBUF):
        j = (NR - 1) * NBUF + b
        pltpu.make_async_copy(
            tab_hbm.at[src_v.at[j]], bufs[b], gsem[b]).wait()
        pltpu.async_copy(bufs[b], acc_sh.at[dst_v.at[j]], ssem[b], add=True)
        count_scatter(j)
    for b in range(NBUF):
        j = (NR - 1) * NBUF + b
        pltpu.make_async_copy(
            bufs[b], acc_sh.at[dst_v.at[j]], ssem[b]).wait()
    if with_count:
        def drain(i, carry):
            pltpu.make_async_copy(
                onesb, cnt_sh.at[dst_v.at[0]], csem).wait()
            return carry
        lax.fori_loop(0, NCHUNK // 2, drain, 0)

    plsc.subcore_barrier()

    nwb = RPT // CH
    for k in range(nwb):
        b = k % NBUF
        r = row0 + k * CH
        if k >= NBUF:
            rp = row0 + (k - NBUF) * CH
            pltpu.make_async_copy(
                bufs[b], agg_hbm.at[pl.ds(rp, CH), cid], gsem[b]).wait()
        pltpu.sync_copy(acc_sh.at[pl.ds(r, CH)], bufs[b])
        pltpu.async_copy(bufs[b], agg_hbm.at[pl.ds(r, CH), cid], gsem[b])
    for k in range(max(0, nwb - NBUF), nwb):
        b = k % NBUF
        r = row0 + k * CH
        pltpu.make_async_copy(
            bufs[b], agg_hbm.at[pl.ds(r, CH), cid], gsem[b]).wait()
    if with_count:
        for k in range(RPT // CH):
            r = row0 + k * CH
            pltpu.sync_copy(cnt_sh.at[pl.ds(r, CH)], cbuf)
            pltpu.sync_copy(cbuf, cnt_hbm.at[cid, pl.ds(r, CH)])


def _make_sc_pass(with_count):
    mesh = plsc.VectorSubcoreMesh(core_axis_name="c", subcore_axis_name="s")
    out_type = [jax.ShapeDtypeStruct((R, NC, DH), jnp.float32)]
    scratch = [
        pltpu.VMEM_SHARED((R, DH), jnp.float32),
    ]
    if with_count:
        out_type.append(jax.ShapeDtypeStruct((NC, R, 16), jnp.float32))
        scratch.append(pltpu.VMEM_SHARED((R, 16), jnp.float32))
    scratch += [
        pltpu.VMEM((NCHUNK, CH), jnp.int32),
        pltpu.VMEM((NCHUNK, CH), jnp.int32),
    ]
    scratch += [pltpu.VMEM((CH, DH), jnp.float32)] * NBUF
    if with_count:
        scratch += [
            pltpu.VMEM((CH, 16), jnp.float32),
            pltpu.VMEM((CH, 16), jnp.float32),
        ]
    scratch += [pltpu.SemaphoreType.DMA] * (2 * NBUF)
    if with_count:
        scratch += [pltpu.SemaphoreType.DMA]
    return pl.kernel(
        functools.partial(_sc_body, with_count),
        out_type=out_type,
        mesh=mesh,
        scratch_types=scratch,
        compiler_params=pltpu.CompilerParams(use_tc_tiling_on_sc=False),
    )


CH2 = 64
NCHUNK2 = 318
EPT2 = CH2 * NCHUNK2
E_PAD2 = EPT2 * NS


def _sc_body2(tab2_hbm, src_hbm, dst_hbm, z2,
              agg_hbm,
              tab_sh, acc_sh, src_v, dst_v, b0, b1,
              g0, g1, s0, s1):
    bufs = (b0, b1)
    gsem = (g0, g1)
    ssem = (s0, s1)

    cid = lax.axis_index("c")
    sid = lax.axis_index("s")
    row0 = sid * RPT

    pltpu.sync_copy(z2, bufs[0])
    for k in range(RPT // CH2):
        pltpu.sync_copy(bufs[0], acc_sh.at[pl.ds(row0 + k * CH2, CH2)])
    for k in range(RPT // CH2):
        r = row0 + k * CH2
        pltpu.sync_copy(tab2_hbm.at[cid, pl.ds(r, CH2)], bufs[1])
        pltpu.sync_copy(bufs[1], tab_sh.at[pl.ds(r, CH2)])

    pltpu.sync_copy(src_hbm.at[sid], src_v)
    pltpu.sync_copy(dst_hbm.at[sid], dst_v)

    plsc.subcore_barrier()

    for b in range(2):
        pltpu.async_copy(tab_sh.at[src_v.at[b]], bufs[b], gsem[b])

    NR = NCHUNK2 // 2

    def round_body(g, carry):
        for b in range(2):
            j = 2 * g + b
            pltpu.make_async_copy(
                tab_sh.at[src_v.at[j]], bufs[b], gsem[b]).wait()
            pltpu.async_copy(bufs[b], acc_sh.at[dst_v.at[j]], ssem[b],
                             add=True)
        for b in range(2):
            j = 2 * g + b
            pltpu.make_async_copy(
                bufs[b], acc_sh.at[dst_v.at[j]], ssem[b]).wait()
            pltpu.async_copy(tab_sh.at[src_v.at[j + 2]], bufs[b], gsem[b])
        return carry

    lax.fori_loop(0, NR - 1, round_body, 0)
    for b in range(2):
        j = (NR - 1) * 2 + b
        pltpu.make_async_copy(
            tab_sh.at[src_v.at[j]], bufs[b], gsem[b]).wait()
        pltpu.async_copy(bufs[b], acc_sh.at[dst_v.at[j]], ssem[b], add=True)
    for b in range(2):
        j = (NR - 1) * 2 + b
        pltpu.make_async_copy(
            bufs[b], acc_sh.at[dst_v.at[j]], ssem[b]).wait()

    plsc.subcore_barrier()

    nwb = RPT // CH2
    for k in range(nwb):
        b = k % 2
        r = row0 + k * CH2
        if k >= 2:
            rp = row0 + (k - 2) * CH2
            pltpu.make_async_copy(
                bufs[b], agg_hbm.at[pl.ds(rp, CH2), cid], gsem[b]).wait()
        pltpu.sync_copy(acc_sh.at[pl.ds(r, CH2)], bufs[b])
        pltpu.async_copy(bufs[b], agg_hbm.at[pl.ds(r, CH2), cid], gsem[b])
    for k in range(nwb - 2, nwb):
        b = k % 2
        r = row0 + k * CH2
        pltpu.make_async_copy(
            bufs[b], agg_hbm.at[pl.ds(r, CH2), cid], gsem[b]).wait()


def _make_sc_pass2():
    mesh = plsc.VectorSubcoreMesh(core_axis_name="c", subcore_axis_name="s")
    return pl.kernel(
        _sc_body2,
        out_type=[jax.ShapeDtypeStruct((R, NC, DH), jnp.float32)],
        mesh=mesh,
        scratch_types=[
            pltpu.VMEM_SHARED((R, DH), jnp.float32),
            pltpu.VMEM_SHARED((R, DH), jnp.float32),
            pltpu.VMEM((NCHUNK2, CH2), jnp.int32),
            pltpu.VMEM((NCHUNK2, CH2), jnp.int32),
            pltpu.VMEM((CH2, DH), jnp.float32),
            pltpu.VMEM((CH2, DH), jnp.float32),
            pltpu.SemaphoreType.DMA, pltpu.SemaphoreType.DMA,
            pltpu.SemaphoreType.DMA, pltpu.SemaphoreType.DMA,
        ],
        compiler_params=pltpu.CompilerParams(use_tc_tiling_on_sc=False),
    )


def _tc_layer_body(agg_ref, cnt_ref, x_ref, wl_ref, wr_ref, b_ref, o_ref):
    c = cnt_ref[0, :, 0] + cnt_ref[1, :, 0]
    mean = agg_ref[...] / jnp.maximum(c, 1.0)[:, None]
    h = (jnp.dot(mean, wl_ref[...], preferred_element_type=jnp.float32)
         + jnp.dot(x_ref[...], wr_ref[...], preferred_element_type=jnp.float32)
         + b_ref[...])
    o_ref[...] = jnp.maximum(h, 0.0)


def _tc_layer(agg, cnt, x, wl, wr, b):
    blk = 640
    return pl.pallas_call(
        _tc_layer_body,
        grid=(R // blk,),
        in_specs=[
            pl.BlockSpec((blk, D), lambda i: (i, 0)),
            pl.BlockSpec((NC, blk, 16), lambda i: (0, i, 0)),
            pl.BlockSpec((blk, D), lambda i: (i, 0)),
            pl.BlockSpec((D, D), lambda i: (0, 0)),
            pl.BlockSpec((D, D), lambda i: (0, 0)),
            pl.BlockSpec((1, D), lambda i: (0, 0)),
        ],
        out_specs=pl.BlockSpec((blk, D), lambda i: (i, 0)),
        out_shape=jax.ShapeDtypeStruct((R, D), jnp.float32),
    )(agg, cnt, x, wl, wr, b)


def _tc_final_body(agg_ref, cnt_ref, h_ref, wl_ref, wr_ref, b_ref,
                   wf_ref, bf_ref, o_ref):
    c = cnt_ref[0, :, 0] + cnt_ref[1, :, 0]
    mean = agg_ref[...] / jnp.maximum(c, 1.0)[:, None]
    h2 = (jnp.dot(mean, wl_ref[...], preferred_element_type=jnp.float32)
          + jnp.dot(h_ref[...], wr_ref[...], preferred_element_type=jnp.float32)
          + b_ref[...])
    h2 = jnp.maximum(h2, 0.0)
    o_ref[...] = (jnp.dot(h2, wf_ref[...], preferred_element_type=jnp.float32)
                  + bf_ref[...])


def _tc_final(agg, cnt, h, wl, wr, b, wf_pad, bf_pad):
    blk = 640
    return pl.pallas_call(
        _tc_final_body,
        grid=(R // blk,),
        in_specs=[
            pl.BlockSpec((blk, D), lambda i: (i, 0)),
            pl.BlockSpec((NC, blk, 16), lambda i: (0, i, 0)),
            pl.BlockSpec((blk, D), lambda i: (i, 0)),
            pl.BlockSpec((D, D), lambda i: (0, 0)),
            pl.BlockSpec((D, D), lambda i: (0, 0)),
            pl.BlockSpec((1, D), lambda i: (0, 0)),
            pl.BlockSpec((D, D), lambda i: (0, 0)),
            pl.BlockSpec((1, D), lambda i: (0, 0)),
        ],
        out_specs=pl.BlockSpec((blk, D), lambda i: (i, 0)),
        out_shape=jax.ShapeDtypeStruct((R, D), jnp.float32),
    )(agg, cnt, h, wl, wr, b, wf_pad, bf_pad)


def kernel(x, edge_index, Wl1, Wr1, b1, Wl2, Wr2, b2, Wfc, bfc):
    src = edge_index[0].astype(jnp.int32)
    dst = edge_index[1].astype(jnp.int32)
    npad = E_PAD - N_EDGES
    pad_ar = jnp.arange(npad, dtype=jnp.int32)
    src_p = jnp.concatenate(
        [src, pad_ar % N_NODES]).reshape(NS, NCHUNK, CH)
    dst_p = jnp.concatenate(
        [dst, N_NODES + pad_ar % (R - N_NODES)]).reshape(NS, NCHUNK, CH)
    src2 = jnp.stack([2 * src_p, 2 * src_p + 1])
    xp = jnp.zeros((R, D), jnp.float32).at[:N_NODES].set(x)

    z64 = jnp.zeros((CH, DH), jnp.float32)
    z16 = jnp.zeros((CH, 16), jnp.float32)
    o16 = jnp.ones((CH, 16), jnp.float32)

    npad2 = E_PAD2 - N_EDGES
    pad_ar2 = jnp.arange(npad2, dtype=jnp.int32)
    srcp2 = jnp.concatenate(
        [src, pad_ar2 % N_NODES]).reshape(NS, NCHUNK2, CH2)
    dstp2 = jnp.concatenate(
        [dst, N_NODES + pad_ar2 % (R - N_NODES)]).reshape(NS, NCHUNK2, CH2)
    z2 = jnp.zeros((CH2, DH), jnp.float32)

    sc1 = _make_sc_pass(True)
    sc2 = _make_sc_pass2()

    agg1, cnt = sc1(xp.reshape(2 * R, DH), src2, dst_p, z64, z16, o16)
    h = _tc_layer(agg1.reshape(R, D), cnt, xp, Wl1, Wr1, b1.reshape(1, D))
    h_split = h.reshape(R, NC, DH).transpose(1, 0, 2)
    (agg2,) = sc2(h_split, srcp2, dstp2, z2)

    wf_pad = jnp.zeros((D, D), jnp.float32).at[:, :1].set(Wfc)
    bf_pad = jnp.broadcast_to(bfc.reshape(1, 1), (1, D))
    outp = _tc_final(agg2.reshape(R, D), cnt, h, Wl2, Wr2, b2.reshape(1, D),
                     wf_pad, bf_pad)
    return outp[:N_NODES, :1]


# speedup vs baseline: 1.1662x; 1.0184x over previous
import functools

import jax
import jax.numpy as jnp
from jax import lax
from jax.experimental import pallas as pl
from jax.experimental.pallas import tpu as pltpu
from jax.experimental.pallas import tpu_sc as plsc

N_NODES = 10000
N_EDGES = 320000
D = 128
DH = D // 2

NC, NS = 2, 16
CH = 128
EPT = 20480
NCHUNK = EPT // CH
E_PAD = EPT * NS
R = 10240
RPT = R // NS


NBUF = 4


def _sc_body(with_count, *refs):
    if with_count:
        (tab_hbm, src_hbm, dst_hbm, z64, z16, o16,
         agg_hbm, cnt_hbm,
         acc_sh, cnt_sh, src_v, dst_v, b0, b1, b2, b3, onesb, cbuf,
         g0, g1, g2, g3, s0, s1, s2, s3, csem) = refs
    else:
        (tab_hbm, src_hbm, dst_hbm, z64,
         agg_hbm,
         acc_sh, src_v, dst_v, b0, b1, b2, b3,
         g0, g1, g2, g3, s0, s1, s2, s3) = refs
    bufs = (b0, b1, b2, b3)
    gsem = (g0, g1, g2, g3)
    ssem = (s0, s1, s2, s3)

    cid = lax.axis_index("c")
    sid = lax.axis_index("s")
    row0 = sid * RPT

    pltpu.sync_copy(z64, bufs[0])
    for k in range(RPT // CH):
        pltpu.sync_copy(bufs[0], acc_sh.at[pl.ds(row0 + k * CH, CH)])
    if with_count:
        pltpu.sync_copy(z16, cbuf)
        for k in range(RPT // CH):
            pltpu.sync_copy(cbuf, cnt_sh.at[pl.ds(row0 + k * CH, CH)])
        pltpu.sync_copy(o16, onesb)

    pltpu.sync_copy(src_hbm.at[cid, sid], src_v)
    pltpu.sync_copy(dst_hbm.at[sid], dst_v)

    plsc.subcore_barrier()

    def count_scatter(j):
        if with_count:
            @pl.when((j < NCHUNK // 2) == (cid == 0))
            def _():
                pltpu.async_copy(onesb, cnt_sh.at[dst_v.at[j]], csem,
                                 add=True)

    for b in range(NBUF):
        pltpu.async_copy(tab_hbm.at[src_v.at[b]], bufs[b], gsem[b])

    NR = NCHUNK // NBUF

    def round_body(g, carry):
        for b in range(NBUF):
            j = g * NBUF + b
            pltpu.make_async_copy(
                tab_hbm.at[src_v.at[j]], bufs[b], gsem[b]).wait()
            pltpu.async_copy(bufs[b], acc_sh.at[dst_v.at[j]], ssem[b],
                             add=True)
            count_scatter(j)
        for b in range(NBUF):
            j = g * NBUF + b
            pltpu.make_async_copy(
                bufs[b], acc_sh.at[dst_v.at[j]], ssem[b]).wait()
            pltpu.async_copy(tab_hbm.at[src_v.at[j + NBUF]], bufs[b],
                             gsem[b])
        return carry

    lax.fori_loop(0, NR - 1, round_body, 0)
    for b in range(NBUF):
        j = (NR - 1) * NBUF + b
        pltpu.make_async_copy(
            tab_hbm.at[src_v.at[j]], bufs[b], gsem[b]).wait()
        pltpu.async_copy(bufs[b], acc_sh.at[dst_v.at[j]], ssem[b], add=True)
        count_scatter(j)
    for b in range(NBUF):
        j = (NR - 1) * NBUF + b
        pltpu.make_async_copy(
            bufs[b], acc_sh.at[dst_v.at[j]], ssem[b]).wait()
    if with_count:
        def drain(i, carry):
            pltpu.make_async_copy(
                onesb, cnt_sh.at[dst_v.at[0]], csem).wait()
            return carry
        lax.fori_loop(0, NCHUNK // 2, drain, 0)

    plsc.subcore_barrier()

    nwb = RPT // CH
    for k in range(nwb):
        b = k % NBUF
        r = row0 + k * CH
        if k >= NBUF:
            rp = row0 + (k - NBUF) * CH
            pltpu.make_async_copy(
                bufs[b], agg_hbm.at[pl.ds(rp, CH), cid], gsem[b]).wait()
        pltpu.sync_copy(acc_sh.at[pl.ds(r, CH)], bufs[b])
        pltpu.async_copy(bufs[b], agg_hbm.at[pl.ds(r, CH), cid], gsem[b])
    for k in range(max(0, nwb - NBUF), nwb):
        b = k % NBUF
        r = row0 + k * CH
        pltpu.make_async_copy(
            bufs[b], agg_hbm.at[pl.ds(r, CH), cid], gsem[b]).wait()
    if with_count:
        for k in range(RPT // CH):
            r = row0 + k * CH
            pltpu.sync_copy(cnt_sh.at[pl.ds(r, CH)], cbuf)
            pltpu.sync_copy(cbuf, cnt_hbm.at[cid, pl.ds(r, CH)])


def _make_sc_pass(with_count):
    mesh = plsc.VectorSubcoreMesh(core_axis_name="c", subcore_axis_name="s")
    out_type = [jax.ShapeDtypeStruct((R, NC, DH), jnp.float32)]
    scratch = [
        pltpu.VMEM_SHARED((R, DH), jnp.float32),
    ]
    if with_count:
        out_type.append(jax.ShapeDtypeStruct((NC, R, 16), jnp.float32))
        scratch.append(pltpu.VMEM_SHARED((R, 16), jnp.float32))
    scratch += [
        pltpu.VMEM((NCHUNK, CH), jnp.int32),
        pltpu.VMEM((NCHUNK, CH), jnp.int32),
    ]
    scratch += [pltpu.VMEM((CH, DH), jnp.float32)] * NBUF
    if with_count:
        scratch += [
            pltpu.VMEM((CH, 16), jnp.float32),
            pltpu.VMEM((CH, 16), jnp.float32),
        ]
    scratch += [pltpu.SemaphoreType.DMA] * (2 * NBUF)
    if with_count:
        scratch += [pltpu.SemaphoreType.DMA]
    return pl.kernel(
        functools.partial(_sc_body, with_count),
        out_type=out_type,
        mesh=mesh,
        scratch_types=scratch,
        compiler_params=pltpu.CompilerParams(use_tc_tiling_on_sc=False),
    )


CH2 = 64
NCHUNK2 = 318
EPT2 = CH2 * NCHUNK2
E_PAD2 = EPT2 * NS


def _sc_body2(tab2_hbm, src_hbm, dst_hbm, z2,
              agg_hbm,
              tab_sh, acc_sh, src_v, dst_v, b0, b1,
              g0, g1, s0, s1):
    bufs = (b0, b1)
    gsem = (g0, g1)
    ssem = (s0, s1)

    cid = lax.axis_index("c")
    sid = lax.axis_index("s")
    row0 = sid * RPT

    pltpu.sync_copy(z2, bufs[0])
    for k in range(RPT // CH2):
        pltpu.sync_copy(bufs[0], acc_sh.at[pl.ds(row0 + k * CH2, CH2)])
    for k in range(RPT // CH2):
        r = row0 + k * CH2
        pltpu.sync_copy(tab2_hbm.at[cid, pl.ds(r, CH2)], bufs[1])
        pltpu.sync_copy(bufs[1], tab_sh.at[pl.ds(r, CH2)])

    pltpu.sync_copy(src_hbm.at[sid], src_v)
    pltpu.sync_copy(dst_hbm.at[sid], dst_v)

    plsc.subcore_barrier()

    for b in range(2):
        pltpu.async_copy(tab_sh.at[src_v.at[b]], bufs[b], gsem[b])

    NR = NCHUNK2 // 2

    def round_body(g, carry):
        for b in range(2):
            j = 2 * g + b
            pltpu.make_async_copy(
                tab_sh.at[src_v.at[j]], bufs[b], gsem[b]).wait()
            pltpu.async_copy(bufs[b], acc_sh.at[dst_v.at[j]], ssem[b],
                             add=True)
        for b in range(2):
            j = 2 * g + b
            pltpu.make_async_copy(
                bufs[b], acc_sh.at[dst_v.at[j]], ssem[b]).wait()
            pltpu.async_copy(tab_sh.at[src_v.at[j + 2]], bufs[b], gsem[b])
        return carry

    lax.fori_loop(0, NR - 1, round_body, 0)
    for b in range(2):
        j = (NR - 1) * 2 + b
        pltpu.make_async_copy(
            tab_sh.at[src_v.at[j]], bufs[b], gsem[b]).wait()
        pltpu.async_copy(bufs[b], acc_sh.at[dst_v.at[j]], ssem[b], add=True)
    for b in range(2):
        j = (NR - 1) * 2 + b
        pltpu.make_async_copy(
            bufs[b], acc_sh.at[dst_v.at[j]], ssem[b]).wait()

    plsc.subcore_barrier()

    nwb = RPT // CH2
    for k in range(nwb):
        b = k % 2
        r = row0 + k * CH2
        if k >= 2:
            rp = row0 + (k - 2) * CH2
            pltpu.make_async_copy(
                bufs[b], agg_hbm.at[pl.ds(rp, CH2), cid], gsem[b]).wait()
        pltpu.sync_copy(acc_sh.at[pl.ds(r, CH2)], bufs[b])
        pltpu.async_copy(bufs[b], agg_hbm.at[pl.ds(r, CH2), cid], gsem[b])
    for k in range(nwb - 2, nwb):
        b = k % 2
        r = row0 + k * CH2
        pltpu.make_async_copy(
            bufs[b], agg_hbm.at[pl.ds(r, CH2), cid], gsem[b]).wait()


def _make_sc_pass2():
    mesh = plsc.VectorSubcoreMesh(core_axis_name="c", subcore_axis_name="s")
    return pl.kernel(
        _sc_body2,
        out_type=[jax.ShapeDtypeStruct((R, NC, DH), jnp.float32)],
        mesh=mesh,
        scratch_types=[
            pltpu.VMEM_SHARED((R, DH), jnp.float32),
            pltpu.VMEM_SHARED((R, DH), jnp.float32),
            pltpu.VMEM((NCHUNK2, CH2), jnp.int32),
            pltpu.VMEM((NCHUNK2, CH2), jnp.int32),
            pltpu.VMEM((CH2, DH), jnp.float32),
            pltpu.VMEM((CH2, DH), jnp.float32),
            pltpu.SemaphoreType.DMA, pltpu.SemaphoreType.DMA,
            pltpu.SemaphoreType.DMA, pltpu.SemaphoreType.DMA,
        ],
        compiler_params=pltpu.CompilerParams(use_tc_tiling_on_sc=False),
    )


def _tc_layer_body(agg_ref, cnt_ref, x_ref, wl_ref, wr_ref, b_ref,
                   o_ref, os_ref):
    c = cnt_ref[0, :, 0] + cnt_ref[1, :, 0]
    mean = agg_ref[...] / jnp.maximum(c, 1.0)[:, None]
    h = (jnp.dot(mean, wl_ref[...], preferred_element_type=jnp.float32)
         + jnp.dot(x_ref[...], wr_ref[...], preferred_element_type=jnp.float32)
         + b_ref[...])
    h = jnp.maximum(h, 0.0)
    o_ref[...] = h
    os_ref[0] = h[:, :DH]
    os_ref[1] = h[:, DH:]


def _tc_layer(agg, cnt, x, wl, wr, b):
    blk = 640
    return pl.pallas_call(
        _tc_layer_body,
        grid=(R // blk,),
        in_specs=[
            pl.BlockSpec((blk, D), lambda i: (i, 0)),
            pl.BlockSpec((NC, blk, 16), lambda i: (0, i, 0)),
            pl.BlockSpec((blk, D), lambda i: (i, 0)),
            pl.BlockSpec((D, D), lambda i: (0, 0)),
            pl.BlockSpec((D, D), lambda i: (0, 0)),
            pl.BlockSpec((1, D), lambda i: (0, 0)),
        ],
        out_specs=[
            pl.BlockSpec((blk, D), lambda i: (i, 0)),
            pl.BlockSpec((NC, blk, DH), lambda i: (0, i, 0)),
        ],
        out_shape=[
            jax.ShapeDtypeStruct((R, D), jnp.float32),
            jax.ShapeDtypeStruct((NC, R, DH), jnp.float32),
        ],
    )(agg, cnt, x, wl, wr, b)


def _tc_final_body(agg_ref, cnt_ref, h_ref, wl_ref, wr_ref, b_ref,
                   wf_ref, bf_ref, o_ref):
    c = cnt_ref[0, :, 0] + cnt_ref[1, :, 0]
    mean = agg_ref[...] / jnp.maximum(c, 1.0)[:, None]
    h2 = (jnp.dot(mean, wl_ref[...], preferred_element_type=jnp.float32)
          + jnp.dot(h_ref[...], wr_ref[...], preferred_element_type=jnp.float32)
          + b_ref[...])
    h2 = jnp.maximum(h2, 0.0)
    o_ref[...] = (jnp.dot(h2, wf_ref[...], preferred_element_type=jnp.float32)
                  + bf_ref[...])


def _tc_final(agg, cnt, h, wl, wr, b, wf_pad, bf_pad):
    blk = 640
    return pl.pallas_call(
        _tc_final_body,
        grid=(R // blk,),
        in_specs=[
            pl.BlockSpec((blk, D), lambda i: (i, 0)),
            pl.BlockSpec((NC, blk, 16), lambda i: (0, i, 0)),
            pl.BlockSpec((blk, D), lambda i: (i, 0)),
            pl.BlockSpec((D, D), lambda i: (0, 0)),
            pl.BlockSpec((D, D), lambda i: (0, 0)),
            pl.BlockSpec((1, D), lambda i: (0, 0)),
            pl.BlockSpec((D, D), lambda i: (0, 0)),
            pl.BlockSpec((1, D), lambda i: (0, 0)),
        ],
        out_specs=pl.BlockSpec((blk, D), lambda i: (i, 0)),
        out_shape=jax.ShapeDtypeStruct((R, D), jnp.float32),
    )(agg, cnt, h, wl, wr, b, wf_pad, bf_pad)


def kernel(x, edge_index, Wl1, Wr1, b1, Wl2, Wr2, b2, Wfc, bfc):
    src = edge_index[0].astype(jnp.int32)
    dst = edge_index[1].astype(jnp.int32)
    npad = E_PAD - N_EDGES
    pad_ar = jnp.arange(npad, dtype=jnp.int32)
    src_p = jnp.concatenate(
        [src, pad_ar % N_NODES]).reshape(NS, NCHUNK, CH)
    dst_p = jnp.concatenate(
        [dst, N_NODES + pad_ar % (R - N_NODES)]).reshape(NS, NCHUNK, CH)
    src2 = jnp.stack([2 * src_p, 2 * src_p + 1])
    xp = jnp.zeros((R, D), jnp.float32).at[:N_NODES].set(x)

    z64 = jnp.zeros((CH, DH), jnp.float32)
    z16 = jnp.zeros((CH, 16), jnp.float32)
    o16 = jnp.ones((CH, 16), jnp.float32)

    npad2 = E_PAD2 - N_EDGES
    pad_ar2 = jnp.arange(npad2, dtype=jnp.int32)
    srcp2 = jnp.concatenate(
        [src, pad_ar2 % N_NODES]).reshape(NS, NCHUNK2, CH2)
    dstp2 = jnp.concatenate(
        [dst, N_NODES + pad_ar2 % (R - N_NODES)]).reshape(NS, NCHUNK2, CH2)
    z2 = jnp.zeros((CH2, DH), jnp.float32)

    sc1 = _make_sc_pass(True)
    sc2 = _make_sc_pass2()

    agg1, cnt = sc1(xp.reshape(2 * R, DH), src2, dst_p, z64, z16, o16)
    h, h_split = _tc_layer(agg1.reshape(R, D), cnt, xp, Wl1, Wr1,
                           b1.reshape(1, D))
    (agg2,) = sc2(h_split, srcp2, dstp2, z2)

    wf_pad = jnp.zeros((D, D), jnp.float32).at[:, :1].set(Wfc)
    bf_pad = jnp.broadcast_to(bfc.reshape(1, 1), (1, D))
    outp = _tc_final(agg2.reshape(R, D), cnt, h, Wl2, Wr2, b2.reshape(1, D),
                     wf_pad, bf_pad)
    return outp[:N_NODES, :1]
